# baseline (speedup 1.0000x reference)
import functools

import jax
import jax.numpy as jnp
from jax import lax
from jax.experimental import pallas as pl
from jax.experimental.pallas import tpu as pltpu
from jax.experimental.pallas import tpu_sc as plsc

NC = 2
NS = 16
NW = NC * NS
CHUNK = 128


@functools.lru_cache(maxsize=None)
def _sc_partial_sums(n_nodes: int, d: int, n_chunks0: int, n_chunks1: int):
    n_chunks = max(n_chunks0, n_chunks1)
    acc_rows = -(-(n_nodes + 1) // (NS * 128)) * (NS * 128)
    zero_chunks_per_tile = acc_rows // NS // 128
    out_rows_per_tile = acc_rows // NS
    assert d % 16 == 0

    mesh = plsc.VectorSubcoreMesh(core_axis_name="c", subcore_axis_name="s")

    @functools.partial(
        pl.kernel,
        out_type=jax.ShapeDtypeStruct((NC, acc_rows, d), jnp.float32),
        mesh=mesh,
        scratch_types=[
            pltpu.VMEM((n_chunks, CHUNK), jnp.int32),
            pltpu.VMEM((n_chunks, CHUNK), jnp.int32),
            pltpu.VMEM((CHUNK, d), jnp.float32),
            pltpu.VMEM_SHARED((acc_rows, d), jnp.float32),
            pltpu.SemaphoreType.DMA,
        ],
    )
    def sc_kernel(x_hbm, col_hbm, row_hbm, out_hbm, col_v, row_v, gbuf, acc, sem):
        cid = lax.axis_index("c")
        sid = lax.axis_index("s")
        wid = cid * NS + sid

        pltpu.sync_copy(col_hbm.at[wid], col_v)
        pltpu.sync_copy(row_hbm.at[wid], row_v)

        def zero_body(i, carry):
            for j in range(d // 16):
                gbuf[i, pl.ds(j * 16, 16)] = jnp.zeros((16,), jnp.float32)
            return carry
        lax.fori_loop(0, CHUNK, zero_body, 0)
        for k in range(zero_chunks_per_tile):
            pltpu.sync_copy(
                gbuf, acc.at[pl.ds((sid * zero_chunks_per_tile + k) * 128, 128)]
            )
        plsc.subcore_barrier()

        def body(j, carry):
            pltpu.async_copy(x_hbm.at[col_v.at[j]], gbuf, sem).wait()
            pltpu.sync_copy(gbuf, acc.at[row_v.at[j]], add=True)
            return carry
        my_chunks = jnp.where(cid == 0, n_chunks0, n_chunks1)
        lax.fori_loop(0, my_chunks, body, 0)
        plsc.subcore_barrier()

        pltpu.sync_copy(
            acc.at[pl.ds(sid * out_rows_per_tile, out_rows_per_tile)],
            out_hbm.at[cid, pl.ds(sid * out_rows_per_tile, out_rows_per_tile)],
        )

    return sc_kernel


@functools.lru_cache(maxsize=None)
def _combine(n_nodes: int, d: int):
    blk = 1000
    assert n_nodes % blk == 0

    def body(x_ref, a_ref, b_ref, o_ref):
        o_ref[:, :d] = x_ref[...]
        o_ref[:, d:] = a_ref[...] + b_ref[...]

    return pl.pallas_call(
        body,
        grid=(n_nodes // blk,),
        in_specs=[pl.BlockSpec((blk, d), lambda i: (i, 0))] * 3,
        out_specs=pl.BlockSpec((blk, 2 * d), lambda i: (i, 0)),
        out_shape=jax.ShapeDtypeStruct((n_nodes, 2 * d), jnp.float32),
    )


FRAC0 = 0.40


def kernel(x, edge_index):
    n_nodes, d = x.shape
    n_edges = edge_index.shape[1]
    ei = edge_index.astype(jnp.int32)
    row, col = ei[0], ei[1]

    total_chunks = -(-n_edges // (NS * CHUNK))
    n0 = max(1, round(total_chunks * FRAC0))
    n1 = total_chunks - n0
    n_max = max(n0, n1)
    e_pad = NS * CHUNK * total_chunks
    if e_pad != n_edges:
        pad = e_pad - n_edges
        row = jnp.concatenate([row, jnp.full((pad,), n_nodes, jnp.int32)])
        col = jnp.concatenate([col, jnp.zeros((pad,), jnp.int32)])

    c0 = NS * n0 * CHUNK
    def layout(a):
        a0 = a[:c0].reshape(NS, n0, CHUNK)
        a1 = a[c0:].reshape(NS, n1, CHUNK)
        a0 = jnp.pad(a0, ((0, 0), (0, n_max - n0), (0, 0)))
        a1 = jnp.pad(a1, ((0, 0), (0, n_max - n1), (0, 0)))
        return jnp.concatenate([a0, a1], axis=0)

    partial = _sc_partial_sums(n_nodes, d, n0, n1)(x, layout(col), layout(row))
    return _combine(n_nodes, d)(x, partial[0, :n_nodes], partial[1, :n_nodes])


# speedup vs baseline: 1.0594x; 1.0594x over previous
import functools

import jax
import jax.numpy as jnp
from jax import lax
from jax.experimental import pallas as pl
from jax.experimental.pallas import tpu as pltpu
from jax.experimental.pallas import tpu_sc as plsc

NC = 2
NS = 16
NW = NC * NS
CHUNK = 128


@functools.lru_cache(maxsize=None)
def _sc_partial_sums(n_nodes: int, d: int, n_chunks0: int, n_chunks1: int):
    n_chunks = max(n_chunks0, n_chunks1)
    acc_rows = -(-(n_nodes + 1) // (NS * 128)) * (NS * 128)
    zero_chunks_per_tile = acc_rows // NS // 128
    out_rows_per_tile = acc_rows // NS
    assert d % 16 == 0

    mesh = plsc.VectorSubcoreMesh(core_axis_name="c", subcore_axis_name="s")

    @functools.partial(
        pl.kernel,
        out_type=jax.ShapeDtypeStruct((NC, acc_rows, d), jnp.float32),
        mesh=mesh,
        scratch_types=[
            pltpu.VMEM((n_chunks, CHUNK), jnp.int32),
            pltpu.VMEM((n_chunks, CHUNK), jnp.int32),
            pltpu.VMEM((CHUNK, d), jnp.float32),
            pltpu.VMEM_SHARED((acc_rows, d), jnp.float32),
            pltpu.SemaphoreType.DMA,
        ],
    )
    def sc_kernel(x_hbm, col_hbm, row_hbm, out_hbm, col_v, row_v, gbuf, acc, sem):
        cid = lax.axis_index("c")
        sid = lax.axis_index("s")
        wid = cid * NS + sid

        pltpu.sync_copy(col_hbm.at[wid], col_v)
        pltpu.sync_copy(row_hbm.at[wid], row_v)

        def zero_body(i, carry):
            for j in range(d // 16):
                gbuf[i, pl.ds(j * 16, 16)] = jnp.zeros((16,), jnp.float32)
            return carry
        lax.fori_loop(0, CHUNK, zero_body, 0)
        for k in range(zero_chunks_per_tile):
            pltpu.sync_copy(
                gbuf, acc.at[pl.ds((sid * zero_chunks_per_tile + k) * 128, 128)]
            )
        plsc.subcore_barrier()

        def body(j, carry):
            pltpu.async_copy(x_hbm.at[col_v.at[j]], gbuf, sem).wait()
            pltpu.sync_copy(gbuf, acc.at[row_v.at[j]], add=True)
            return carry
        my_chunks = jnp.where(cid == 0, n_chunks0, n_chunks1)
        lax.fori_loop(0, my_chunks, body, 0)
        plsc.subcore_barrier()

        pltpu.sync_copy(
            acc.at[pl.ds(sid * out_rows_per_tile, out_rows_per_tile)],
            out_hbm.at[cid, pl.ds(sid * out_rows_per_tile, out_rows_per_tile)],
        )

    return sc_kernel


@functools.lru_cache(maxsize=None)
def _combine(n_nodes: int, d: int):
    blk = 1000
    assert n_nodes % blk == 0

    def body(x_ref, a_ref, b_ref, o_ref):
        o_ref[:, :d] = x_ref[...]
        o_ref[:, d:] = a_ref[...] + b_ref[...]

    return pl.pallas_call(
        body,
        grid=(n_nodes // blk,),
        in_specs=[pl.BlockSpec((blk, d), lambda i: (i, 0))] * 3,
        out_specs=pl.BlockSpec((blk, 2 * d), lambda i: (i, 0)),
        out_shape=jax.ShapeDtypeStruct((n_nodes, 2 * d), jnp.float32),
    )


FRAC0 = 0.48


def kernel(x, edge_index):
    n_nodes, d = x.shape
    n_edges = edge_index.shape[1]
    ei = edge_index.astype(jnp.int32)
    row, col = ei[0], ei[1]

    total_chunks = -(-n_edges // (NS * CHUNK))
    n0 = max(1, round(total_chunks * FRAC0))
    n1 = total_chunks - n0
    n_max = max(n0, n1)
    e_pad = NS * CHUNK * total_chunks
    if e_pad != n_edges:
        pad = e_pad - n_edges
        row = jnp.concatenate([row, jnp.full((pad,), n_nodes, jnp.int32)])
        col = jnp.concatenate([col, jnp.zeros((pad,), jnp.int32)])

    c0 = NS * n0 * CHUNK
    def layout(a):
        a0 = a[:c0].reshape(NS, n0, CHUNK)
        a1 = a[c0:].reshape(NS, n1, CHUNK)
        a0 = jnp.pad(a0, ((0, 0), (0, n_max - n0), (0, 0)))
        a1 = jnp.pad(a1, ((0, 0), (0, n_max - n1), (0, 0)))
        return jnp.concatenate([a0, a1], axis=0)

    partial = _sc_partial_sums(n_nodes, d, n0, n1)(x, layout(col), layout(row))
    return _combine(n_nodes, d)(x, partial[0, :n_nodes], partial[1, :n_nodes])


# speedup vs baseline: 1.0764x; 1.0161x over previous
import functools

import jax
import jax.numpy as jnp
from jax import lax
from jax.experimental import pallas as pl
from jax.experimental.pallas import tpu as pltpu
from jax.experimental.pallas import tpu_sc as plsc

NC = 2
NS = 16
NW = NC * NS
CHUNK = 128


@functools.lru_cache(maxsize=None)
def _sc_partial_sums(n_nodes: int, d: int, n_chunks0: int, n_chunks1: int):
    n_chunks = max(n_chunks0, n_chunks1)
    acc_rows = -(-(n_nodes + 1) // (NS * 128)) * (NS * 128)
    zero_chunks_per_tile = acc_rows // NS // 128
    out_rows_per_tile = acc_rows // NS
    assert d % 16 == 0

    mesh = plsc.VectorSubcoreMesh(core_axis_name="c", subcore_axis_name="s")

    @functools.partial(
        pl.kernel,
        out_type=jax.ShapeDtypeStruct((NC, acc_rows, d), jnp.float32),
        mesh=mesh,
        scratch_types=[
            pltpu.VMEM((n_chunks, CHUNK), jnp.int32),
            pltpu.VMEM((n_chunks, CHUNK), jnp.int32),
            pltpu.VMEM((CHUNK, d), jnp.float32),
            pltpu.VMEM_SHARED((acc_rows, d), jnp.float32),
            pltpu.SemaphoreType.DMA,
        ],
    )
    def sc_kernel(x_hbm, col_hbm, row_hbm, out_hbm, col_v, row_v, gbuf, acc, sem):
        cid = lax.axis_index("c")
        sid = lax.axis_index("s")
        wid = cid * NS + sid

        pltpu.sync_copy(col_hbm.at[wid], col_v)
        pltpu.sync_copy(row_hbm.at[wid], row_v)

        def zero_body(i, carry):
            for j in range(d // 16):
                gbuf[i, pl.ds(j * 16, 16)] = jnp.zeros((16,), jnp.float32)
            return carry
        lax.fori_loop(0, CHUNK, zero_body, 0)
        for k in range(zero_chunks_per_tile):
            pltpu.sync_copy(
                gbuf, acc.at[pl.ds((sid * zero_chunks_per_tile + k) * 128, 128)]
            )
        plsc.subcore_barrier()

        def body(j, carry):
            pltpu.async_copy(x_hbm.at[col_v.at[j]], gbuf, sem).wait()
            pltpu.sync_copy(gbuf, acc.at[row_v.at[j]], add=True)
            return carry
        my_chunks = jnp.where(cid == 0, n_chunks0, n_chunks1)
        lax.fori_loop(0, my_chunks, body, 0)
        plsc.subcore_barrier()

        pltpu.sync_copy(
            acc.at[pl.ds(sid * out_rows_per_tile, out_rows_per_tile)],
            out_hbm.at[cid, pl.ds(sid * out_rows_per_tile, out_rows_per_tile)],
        )

    return sc_kernel


@functools.lru_cache(maxsize=None)
def _combine(n_nodes: int, d: int):
    blk = 1000
    assert n_nodes % blk == 0

    def body(x_ref, a_ref, b_ref, o_ref):
        o_ref[:, :d] = x_ref[...]
        o_ref[:, d:] = a_ref[...] + b_ref[...]

    return pl.pallas_call(
        body,
        grid=(n_nodes // blk,),
        in_specs=[pl.BlockSpec((blk, d), lambda i: (i, 0))] * 3,
        out_specs=pl.BlockSpec((blk, 2 * d), lambda i: (i, 0)),
        out_shape=jax.ShapeDtypeStruct((n_nodes, 2 * d), jnp.float32),
    )


FRAC0 = 0.50


def kernel(x, edge_index):
    n_nodes, d = x.shape
    n_edges = edge_index.shape[1]
    ei = edge_index.astype(jnp.int32)
    row, col = ei[0], ei[1]

    total_chunks = -(-n_edges // (NS * CHUNK))
    n0 = max(1, round(total_chunks * FRAC0))
    n1 = total_chunks - n0
    n_max = max(n0, n1)
    e_pad = NS * CHUNK * total_chunks
    if e_pad != n_edges:
        pad = e_pad - n_edges
        row = jnp.concatenate([row, jnp.full((pad,), n_nodes, jnp.int32)])
        col = jnp.concatenate([col, jnp.zeros((pad,), jnp.int32)])

    c0 = NS * n0 * CHUNK
    def layout(a):
        a0 = a[:c0].reshape(NS, n0, CHUNK)
        a1 = a[c0:].reshape(NS, n1, CHUNK)
        a0 = jnp.pad(a0, ((0, 0), (0, n_max - n0), (0, 0)))
        a1 = jnp.pad(a1, ((0, 0), (0, n_max - n1), (0, 0)))
        return jnp.concatenate([a0, a1], axis=0)

    partial = _sc_partial_sums(n_nodes, d, n0, n1)(x, layout(col), layout(row))
    return _combine(n_nodes, d)(x, partial[0, :n_nodes], partial[1, :n_nodes])


# speedup vs baseline: 1.1559x; 1.0738x over previous
import functools

import jax
import jax.numpy as jnp
from jax import lax
from jax.experimental import pallas as pl
from jax.experimental.pallas import tpu as pltpu
from jax.experimental.pallas import tpu_sc as plsc

NC = 2
NS = 16
NW = NC * NS
CHUNK = 128


@functools.lru_cache(maxsize=None)
def _sc_partial_sums(n_nodes: int, d: int, n_chunks0: int, n_chunks1: int):
    n_chunks = max(n_chunks0, n_chunks1)
    acc_rows = -(-(n_nodes + 1) // (NS * 128)) * (NS * 128)
    zero_chunks_per_tile = acc_rows // NS // 128
    out_rows_per_tile = acc_rows // NS
    assert d % 16 == 0

    mesh = plsc.VectorSubcoreMesh(core_axis_name="c", subcore_axis_name="s")

    @functools.partial(
        pl.kernel,
        out_type=jax.ShapeDtypeStruct((NC, acc_rows, d), jnp.float32),
        mesh=mesh,
        scratch_types=[
            pltpu.VMEM((n_chunks, CHUNK), jnp.int32),
            pltpu.VMEM((n_chunks, CHUNK), jnp.int32),
            pltpu.VMEM((CHUNK, d), jnp.float32),
            pltpu.VMEM_SHARED((acc_rows, d), jnp.float32),
            pltpu.SemaphoreType.DMA,
        ],
    )
    def sc_kernel(x_hbm, col0_hbm, col1_hbm, row0_hbm, row1_hbm, out_hbm,
                  col_v, row_v, gbuf, acc, sem):
        cid = lax.axis_index("c")
        sid = lax.axis_index("s")

        @pl.when(cid == 0)
        def _():
            pltpu.sync_copy(col0_hbm.at[sid], col_v.at[pl.ds(0, n_chunks0)])
            pltpu.sync_copy(row0_hbm.at[sid], row_v.at[pl.ds(0, n_chunks0)])

        @pl.when(cid == 1)
        def _():
            pltpu.sync_copy(col1_hbm.at[sid], col_v.at[pl.ds(0, n_chunks1)])
            pltpu.sync_copy(row1_hbm.at[sid], row_v.at[pl.ds(0, n_chunks1)])

        def zero_body(i, carry):
            for j in range(d // 16):
                gbuf[i, pl.ds(j * 16, 16)] = jnp.zeros((16,), jnp.float32)
            return carry
        lax.fori_loop(0, CHUNK, zero_body, 0)
        for k in range(zero_chunks_per_tile):
            pltpu.sync_copy(
                gbuf, acc.at[pl.ds((sid * zero_chunks_per_tile + k) * 128, 128)]
            )
        plsc.subcore_barrier()

        def body(j, carry):
            pltpu.async_copy(x_hbm.at[col_v.at[j]], gbuf, sem).wait()
            pltpu.sync_copy(gbuf, acc.at[row_v.at[j]], add=True)
            return carry
        my_chunks = jnp.where(cid == 0, n_chunks0, n_chunks1)
        lax.fori_loop(0, my_chunks, body, 0)
        plsc.subcore_barrier()

        pltpu.sync_copy(
            acc.at[pl.ds(sid * out_rows_per_tile, out_rows_per_tile)],
            out_hbm.at[cid, pl.ds(sid * out_rows_per_tile, out_rows_per_tile)],
        )

    return sc_kernel


@functools.lru_cache(maxsize=None)
def _combine(n_nodes: int, d: int):
    blk = 1000
    assert n_nodes % blk == 0

    def body(x_ref, a_ref, b_ref, o_ref):
        o_ref[:, :d] = x_ref[...]
        o_ref[:, d:] = a_ref[...] + b_ref[...]

    return pl.pallas_call(
        body,
        grid=(n_nodes // blk,),
        in_specs=[pl.BlockSpec((blk, d), lambda i: (i, 0))] * 3,
        out_specs=pl.BlockSpec((blk, 2 * d), lambda i: (i, 0)),
        out_shape=jax.ShapeDtypeStruct((n_nodes, 2 * d), jnp.float32),
    )


FRAC0 = 0.58


def kernel(x, edge_index):
    n_nodes, d = x.shape
    n_edges = edge_index.shape[1]
    ei = edge_index.astype(jnp.int32)
    row, col = ei[0], ei[1]

    total_chunks = -(-n_edges // (NS * CHUNK))
    n0 = max(1, round(total_chunks * FRAC0))
    n1 = total_chunks - n0
    e_pad = NS * CHUNK * total_chunks
    if e_pad != n_edges:
        pad = e_pad - n_edges
        row = jnp.concatenate([row, jnp.full((pad,), n_nodes, jnp.int32)])
        col = jnp.concatenate([col, jnp.zeros((pad,), jnp.int32)])

    c0 = NS * n0 * CHUNK
    partial = _sc_partial_sums(n_nodes, d, n0, n1)(
        x,
        col[:c0].reshape(NS, n0, CHUNK), col[c0:].reshape(NS, n1, CHUNK),
        row[:c0].reshape(NS, n0, CHUNK), row[c0:].reshape(NS, n1, CHUNK),
    )
    return _combine(n_nodes, d)(x, partial[0, :n_nodes], partial[1, :n_nodes])
